# baseline (speedup 1.0000x reference)
import functools

import jax
import jax.numpy as jnp
from jax import lax
from jax.experimental import pallas as pl
from jax.experimental.pallas import tpu as pltpu
from jax.experimental.pallas import tpu_sc as plsc

NUM_CORES = 2
NUM_SUBCORES = 16
NUM_WORKERS = NUM_CORES * NUM_SUBCORES
EMBED = 32
BT = 128
BT_PER_W = 4
SEQ = 200
CTILES = EMBED // 8


def _body(table_hbm, idx_hbm, out_hbm, idxblk, idx_t, g, s,
          sem_i, sem_g, sem_o):
    wid = lax.axis_index("s") * NUM_CORES + lax.axis_index("c")
    lanes = lax.iota(jnp.int32, 16)

    def transpose_idx(t, carry):
        tv = jnp.full((16,), t, jnp.int32)
        for k in range(8):
            rows = lanes + (k * 16)
            idx_t[t, pl.ds(k * 16, 16)] = plsc.load_gather(
                idxblk, [rows, tv])
        return carry

    def gather_copy(t, p):
        return pltpu.make_async_copy(table_hbm.at[idx_t.at[t]], g.at[p],
                                     sem_g.at[p])

    def out_copy(t, ct, btg, p):
        return pltpu.make_async_copy(
            s.at[p, pl.ds(ct * 8, 8)], out_hbm.at[t, ct, btg], sem_o.at[p])

    def do_t(t, p, btg):
        @pl.when(t >= 2)
        def _():
            for ct in range(CTILES):
                out_copy(t, ct, btg, p).wait()

        gather_copy(t, p).wait()

        @pl.when(t + 1 < SEQ)
        def _():
            gather_copy(t + 1, 1 - p).start()

        for c in range(EMBED):
            cv = jnp.full((16,), c, jnp.int32)
            for k in range(8):
                rows = lanes + (k * 16)
                s[p, c, pl.ds(k * 16, 16)] = plsc.load_gather(
                    g.at[p], [rows, cv])

        for ct in range(CTILES):
            out_copy(t, ct, btg, p).start()

    def do_btile(bt, carry):
        btg = wid * BT_PER_W + bt
        cp_i = pltpu.make_async_copy(
            idx_hbm.at[pl.ds(btg * BT, BT)], idxblk, sem_i)
        cp_i.start()
        cp_i.wait()
        lax.fori_loop(0, SEQ, transpose_idx, 0)
        gather_copy(0, 0).start()

        def step(to, carry):
            for b in range(2):
                do_t(to * 2 + b, b, btg)
            return carry

        lax.fori_loop(0, SEQ // 2, step, 0)
        for ct in range(CTILES):
            out_copy(SEQ - 2, ct, btg, 0).wait()
        for ct in range(CTILES):
            out_copy(SEQ - 1, ct, btg, 1).wait()
        return carry

    lax.fori_loop(0, BT_PER_W, do_btile, 0)


@jax.jit
def _gather(table, idx):
    mesh = plsc.VectorSubcoreMesh(
        core_axis_name="c", subcore_axis_name="s",
        num_cores=NUM_CORES, num_subcores=NUM_SUBCORES)
    return pl.kernel(
        _body,
        out_type=jax.ShapeDtypeStruct((SEQ, CTILES, 128, 8, 128),
                                      jnp.float32),
        mesh=mesh,
        scratch_types=[
            pltpu.VMEM((BT, SEQ), jnp.int32),
            pltpu.VMEM((SEQ, BT), jnp.int32),
            pltpu.VMEM((2, BT, EMBED), jnp.float32),
            pltpu.VMEM((2, EMBED, BT), jnp.float32),
            pltpu.SemaphoreType.DMA,
            pltpu.SemaphoreType.DMA((2,)),
            pltpu.SemaphoreType.DMA((2,)),
        ],
        compiler_params=pltpu.CompilerParams(use_tc_tiling_on_sc=False,
                                             needs_layout_passes=False),
    )(table, idx)


def kernel(regimes, table):
    b, t = regimes.shape
    p = _gather(table, regimes.astype(jnp.int32))
    return jnp.transpose(p, (2, 4, 0, 1, 3)).reshape(b, t, EMBED)


# speedup vs baseline: 3.2509x; 3.2509x over previous
import functools

import jax
import jax.numpy as jnp
from jax import lax
from jax.experimental import pallas as pl
from jax.experimental.pallas import tpu as pltpu
from jax.experimental.pallas import tpu_sc as plsc

NUM_CORES = 2
NUM_SUBCORES = 16
NUM_WORKERS = NUM_CORES * NUM_SUBCORES
EMBED = 32
BT = 128
BT_PER_W = 4
SEQ = 200
CTILES = EMBED // 8


def _body(table_hbm, idx_hbm, out_hbm, idxblk, idx_t, g, s,
          sem_i, sem_g, sem_o):
    wid = lax.axis_index("s") * NUM_CORES + lax.axis_index("c")
    lanes = lax.iota(jnp.int32, 16)

    def transpose_idx(t, carry):
        tv = jnp.full((16,), t, jnp.int32)
        for k in range(8):
            rows = lanes + (k * 16)
            idx_t[t, pl.ds(k * 16, 16)] = plsc.load_gather(
                idxblk, [rows, tv])
        return carry

    def gather_copy(t, p):
        return pltpu.make_async_copy(table_hbm.at[idx_t.at[t]], g.at[p],
                                     sem_g.at[p])

    def out_copy(t, ct, btg, p):
        return pltpu.make_async_copy(
            s.at[p, pl.ds(ct * 8, 8), pl.ds(0, 128)],
            out_hbm.at[t, ct, btg], sem_o.at[p])

    def do_t(t, p, btg):
        @pl.when(t >= 2)
        def _():
            for ct in range(CTILES):
                out_copy(t, ct, btg, p).wait()

        gather_copy(t, p).wait()

        @pl.when(t + 1 < SEQ)
        def _():
            gather_copy(t + 1, 1 - p).start()

        hi = lanes + 16

        def trow(bo, carry):
            for bi in range(8):
                b = bo * 8 + bi
                bv = jnp.full((16,), b, jnp.int32)
                plsc.store_scatter(s.at[p], [lanes, bv],
                                   g[p, b, pl.ds(0, 16)])
                plsc.store_scatter(s.at[p], [hi, bv],
                                   g[p, b, pl.ds(16, 16)])
            return carry

        lax.fori_loop(0, BT // 8, trow, 0)

        for ct in range(CTILES):
            out_copy(t, ct, btg, p).start()

    def do_btile(bt, carry):
        btg = wid * BT_PER_W + bt
        cp_i = pltpu.make_async_copy(
            idx_hbm.at[pl.ds(btg * BT, BT)], idxblk, sem_i)
        cp_i.start()
        cp_i.wait()
        lax.fori_loop(0, SEQ, transpose_idx, 0)
        gather_copy(0, 0).start()

        def step(to, carry):
            for b in range(2):
                do_t(to * 2 + b, b, btg)
            return carry

        lax.fori_loop(0, SEQ // 2, step, 0)
        for ct in range(CTILES):
            out_copy(SEQ - 2, ct, btg, 0).wait()
        for ct in range(CTILES):
            out_copy(SEQ - 1, ct, btg, 1).wait()
        return carry

    lax.fori_loop(0, BT_PER_W, do_btile, 0)


@jax.jit
def _gather(table, idx):
    mesh = plsc.VectorSubcoreMesh(
        core_axis_name="c", subcore_axis_name="s",
        num_cores=NUM_CORES, num_subcores=NUM_SUBCORES)
    return pl.kernel(
        _body,
        out_type=jax.ShapeDtypeStruct((SEQ, CTILES, 128, 8, 128),
                                      jnp.float32),
        mesh=mesh,
        scratch_types=[
            pltpu.VMEM((BT, SEQ), jnp.int32),
            pltpu.VMEM((SEQ, BT), jnp.int32),
            pltpu.VMEM((2, BT, EMBED), jnp.float32),
            pltpu.VMEM((2, EMBED, 129), jnp.float32),
            pltpu.SemaphoreType.DMA,
            pltpu.SemaphoreType.DMA((2,)),
            pltpu.SemaphoreType.DMA((2,)),
        ],
        compiler_params=pltpu.CompilerParams(use_tc_tiling_on_sc=False,
                                             needs_layout_passes=False),
    )(table, idx)


def kernel(regimes, table):
    b, t = regimes.shape
    p = _gather(table, regimes.astype(jnp.int32))
    return jnp.transpose(p, (2, 4, 0, 1, 3)).reshape(b, t, EMBED)


# speedup vs baseline: 3.2988x; 1.0147x over previous
import functools

import jax
import jax.numpy as jnp
from jax import lax
from jax.experimental import pallas as pl
from jax.experimental.pallas import tpu as pltpu
from jax.experimental.pallas import tpu_sc as plsc

NUM_CORES = 2
NUM_SUBCORES = 16
NUM_WORKERS = NUM_CORES * NUM_SUBCORES
EMBED = 32
BT = 128
BT_PER_W = 4
SEQ = 200
CTILES = EMBED // 8
NG = 4


def _body(table_hbm, idx_hbm, out_hbm, idxblk, idx_t, g, s,
          sem_i, sem_g, sem_o):
    wid = lax.axis_index("s") * NUM_CORES + lax.axis_index("c")
    lanes = lax.iota(jnp.int32, 16)

    def transpose_idx(t, carry):
        tv = jnp.full((16,), t, jnp.int32)
        for k in range(8):
            rows = lanes + (k * 16)
            idx_t[t, pl.ds(k * 16, 16)] = plsc.load_gather(
                idxblk, [rows, tv])
        return carry

    def gather_copy(t, p):
        return pltpu.make_async_copy(table_hbm.at[idx_t.at[t]], g.at[p],
                                     sem_g.at[p])

    def out_copy(t, ct, btg, p):
        return pltpu.make_async_copy(
            s.at[p, pl.ds(ct * 8, 8), pl.ds(0, 128)],
            out_hbm.at[t, ct, btg], sem_o.at[p])

    def do_t(t, pg, ps, btg):
        @pl.when(t >= 2)
        def _():
            for ct in range(CTILES):
                out_copy(t, ct, btg, ps).wait()

        gather_copy(t, pg).wait()

        @pl.when(t + 2 < SEQ)
        def _():
            gather_copy(t + 2, (pg + 2) % NG).start()

        hi = lanes + 16

        def trow(bo, carry):
            for bi in range(8):
                b = bo * 8 + bi
                bv = jnp.full((16,), b, jnp.int32)
                plsc.store_scatter(s.at[ps], [lanes, bv],
                                   g[pg, b, pl.ds(0, 16)])
                plsc.store_scatter(s.at[ps], [hi, bv],
                                   g[pg, b, pl.ds(16, 16)])
            return carry

        lax.fori_loop(0, BT // 8, trow, 0)

        for ct in range(CTILES):
            out_copy(t, ct, btg, ps).start()

    def do_btile(bt, carry):
        btg = wid * BT_PER_W + bt
        cp_i = pltpu.make_async_copy(
            idx_hbm.at[pl.ds(btg * BT, BT)], idxblk, sem_i)
        cp_i.start()
        cp_i.wait()
        lax.fori_loop(0, SEQ, transpose_idx, 0)
        gather_copy(0, 0).start()
        gather_copy(1, 1).start()

        def step(to, carry):
            for b in range(NG):
                t = to * NG + b
                do_t(t, b, t % 2, btg)
            return carry

        lax.fori_loop(0, SEQ // NG, step, 0)
        for ct in range(CTILES):
            out_copy(SEQ - 2, ct, btg, 0).wait()
        for ct in range(CTILES):
            out_copy(SEQ - 1, ct, btg, 1).wait()
        return carry

    lax.fori_loop(0, BT_PER_W, do_btile, 0)


@jax.jit
def _gather(table, idx):
    mesh = plsc.VectorSubcoreMesh(
        core_axis_name="c", subcore_axis_name="s",
        num_cores=NUM_CORES, num_subcores=NUM_SUBCORES)
    return pl.kernel(
        _body,
        out_type=jax.ShapeDtypeStruct((SEQ, CTILES, 128, 8, 128),
                                      jnp.float32),
        mesh=mesh,
        scratch_types=[
            pltpu.VMEM((BT, SEQ), jnp.int32),
            pltpu.VMEM((SEQ, BT), jnp.int32),
            pltpu.VMEM((NG, BT, EMBED), jnp.float32),
            pltpu.VMEM((2, EMBED, 129), jnp.float32),
            pltpu.SemaphoreType.DMA,
            pltpu.SemaphoreType.DMA((NG,)),
            pltpu.SemaphoreType.DMA((2,)),
        ],
        compiler_params=pltpu.CompilerParams(use_tc_tiling_on_sc=False,
                                             needs_layout_passes=False),
    )(table, idx)


def kernel(regimes, table):
    b, t = regimes.shape
    p = _gather(table, regimes.astype(jnp.int32))
    return jnp.transpose(p, (2, 4, 0, 1, 3)).reshape(b, t, EMBED)
